# baseline (speedup 1.0000x reference)
import jax
import jax.numpy as jnp
from jax import lax
from jax.experimental import pallas as pl
from jax.experimental.pallas import tpu as pltpu
from jax.experimental.pallas import tpu_sc as plsc

VOCAB = 1000000
D = 64
HIST = 50
BATCH = 4096
NUM_LABELS = 2
DV = 16
NREG = 8

TBLK = 128
NBLK = 977
REG = NBLK * TBLK
VPAD = NREG * REG

NC = 2
NS = 16
NW = NC * NS

BAGS_PER_W = BATCH // NW
IDX_PER_W = BAGS_PER_W * HIST


def _tc_build_table(x0, x1, x2, x3, x4, x5, x6, x7, w_ref, b_ref, o_ref):
    w = w_ref[...]
    parts = [
        lax.dot_general(x[...], w, (((0,), (1,)), ((), ())),
                        preferred_element_type=jnp.float32)
        for x in (x0, x1, x2, x3, x4, x5, x6, x7)
    ]
    o_ref[...] = jnp.concatenate(parts, axis=1) + b_ref[...]


def _sc_bag_sums(feat_hbm, table_hbm, out_hbm, idx_v, rows_v, sums_v, sem):
    wid = lax.axis_index("s") * NC + lax.axis_index("c")

    pltpu.sync_copy(feat_hbm.at[pl.ds(wid * IDX_PER_W, IDX_PER_W)], idx_v)
    pltpu.async_copy(table_hbm.at[idx_v], rows_v, sem).wait()

    zero = jnp.zeros((DV,), jnp.float32)

    def bag_body(bg, _):
        def r_body(r, acc):
            return acc + rows_v[bg * HIST + r, pl.ds(0, DV)]

        sums_v[bg, pl.ds(0, DV)] = lax.fori_loop(0, HIST, r_body, zero)
        return 0

    lax.fori_loop(0, BAGS_PER_W, bag_body, 0)

    pltpu.sync_copy(sums_v, out_hbm.at[pl.ds(wid * BAGS_PER_W, BAGS_PER_W)])


@jax.jit
def _run(features, emb_table, W, b):
    f = features.astype(jnp.int32).reshape(BATCH * HIST)
    gidx = (f % REG) * NREG + f // REG

    w16 = jnp.zeros((DV, D), jnp.float32).at[:NUM_LABELS].set(W) * (1.0 / HIST)
    b128 = jnp.tile(
        jnp.zeros((1, DV), jnp.float32).at[0, :NUM_LABELS].set(b) * (1.0 / HIST),
        (1, NREG))

    region_spec = [
        pl.BlockSpec((D, TBLK), lambda i, k=k: (0, k * NBLK + i))
        for k in range(NREG)
    ]
    vtab = pl.pallas_call(
        _tc_build_table,
        grid=(NBLK,),
        in_specs=region_spec + [
            pl.BlockSpec((DV, D), lambda i: (0, 0)),
            pl.BlockSpec((1, NREG * DV), lambda i: (0, 0)),
        ],
        out_specs=pl.BlockSpec((TBLK, NREG * DV), lambda i: (i, 0)),
        out_shape=jax.ShapeDtypeStruct((REG, NREG * DV), jnp.float32),
    )(*([emb_table.T] * NREG), w16, b128)
    vtab = vtab.reshape(VPAD, DV)

    mesh = plsc.VectorSubcoreMesh(core_axis_name="c", subcore_axis_name="s",
                                  num_cores=NC, num_subcores=NS)
    sums = pl.kernel(
        _sc_bag_sums,
        out_type=jax.ShapeDtypeStruct((BATCH, DV), jnp.float32),
        mesh=mesh,
        compiler_params=pltpu.CompilerParams(use_tc_tiling_on_sc=False),
        scratch_types=[
            pltpu.VMEM((IDX_PER_W,), jnp.int32),
            pltpu.VMEM((IDX_PER_W, DV), jnp.float32),
            pltpu.VMEM((BAGS_PER_W, DV), jnp.float32),
            pltpu.SemaphoreType.DMA,
        ],
    )(gidx, vtab)

    return sums[:, :NUM_LABELS]


def kernel(features, emb_table, W, b):
    return _run(features, emb_table, W, b.astype(jnp.float32))


# speedup vs baseline: 2.1632x; 2.1632x over previous
import jax
import jax.numpy as jnp
from jax import lax
from jax.experimental import pallas as pl
from jax.experimental.pallas import tpu as pltpu
from jax.experimental.pallas import tpu_sc as plsc

VOCAB = 1000000
D = 64
HIST = 50
BATCH = 4096
NUM_LABELS = 2
DV = 16
NREG = 8

TBLK = 1024
NBLK = 123
REG = NBLK * TBLK
VPAD = NREG * REG

NC = 2
NS = 16
NW = NC * NS

BAGS_PER_W = BATCH // NW
IDX_PER_W = BAGS_PER_W * HIST


def _tc_build_table(x0, x1, x2, x3, x4, x5, x6, x7, w_ref, b_ref, o_ref):
    w = w_ref[...]
    parts = [
        lax.dot_general(x[...], w, (((0,), (1,)), ((), ())),
                        preferred_element_type=jnp.float32)
        for x in (x0, x1, x2, x3, x4, x5, x6, x7)
    ]
    o_ref[...] = jnp.concatenate(parts, axis=1) + b_ref[...]


def _sc_bag_sums(feat_hbm, table_hbm, out_hbm, idx_v, rows_v, sums_v, sem):
    wid = lax.axis_index("s") * NC + lax.axis_index("c")

    pltpu.sync_copy(feat_hbm.at[pl.ds(wid * IDX_PER_W, IDX_PER_W)], idx_v)
    pltpu.async_copy(table_hbm.at[idx_v], rows_v, sem).wait()

    zero = jnp.zeros((DV,), jnp.float32)

    def bag_body(bg, _):
        def r_body(r, acc):
            return acc + rows_v[bg * HIST + r, pl.ds(0, DV)]

        sums_v[bg, pl.ds(0, DV)] = lax.fori_loop(0, HIST, r_body, zero)
        return 0

    lax.fori_loop(0, BAGS_PER_W, bag_body, 0)

    pltpu.sync_copy(sums_v, out_hbm.at[pl.ds(wid * BAGS_PER_W, BAGS_PER_W)])


@jax.jit
def _run(features, emb_table, W, b):
    f = features.astype(jnp.int32).reshape(BATCH * HIST)
    gidx = (f % REG) * NREG + f // REG

    w16 = jnp.zeros((DV, D), jnp.float32).at[:NUM_LABELS].set(W) * (1.0 / HIST)
    b128 = jnp.tile(
        jnp.zeros((1, DV), jnp.float32).at[0, :NUM_LABELS].set(b) * (1.0 / HIST),
        (1, NREG))

    maxb = (VOCAB - 1) // TBLK
    region_spec = [
        pl.BlockSpec((D, TBLK),
                     lambda i, k=k: (0, jnp.minimum(k * NBLK + i, maxb)))
        for k in range(NREG)
    ]
    vtab = pl.pallas_call(
        _tc_build_table,
        grid=(NBLK,),
        in_specs=region_spec + [
            pl.BlockSpec((DV, D), lambda i: (0, 0)),
            pl.BlockSpec((1, NREG * DV), lambda i: (0, 0)),
        ],
        out_specs=pl.BlockSpec((TBLK, NREG * DV), lambda i: (i, 0)),
        out_shape=jax.ShapeDtypeStruct((REG, NREG * DV), jnp.float32),
    )(*([emb_table.T] * NREG), w16, b128)
    vtab = vtab.reshape(VPAD, DV)

    mesh = plsc.VectorSubcoreMesh(core_axis_name="c", subcore_axis_name="s",
                                  num_cores=NC, num_subcores=NS)
    sums = pl.kernel(
        _sc_bag_sums,
        out_type=jax.ShapeDtypeStruct((BATCH, DV), jnp.float32),
        mesh=mesh,
        compiler_params=pltpu.CompilerParams(use_tc_tiling_on_sc=False),
        scratch_types=[
            pltpu.VMEM((IDX_PER_W,), jnp.int32),
            pltpu.VMEM((IDX_PER_W, DV), jnp.float32),
            pltpu.VMEM((BAGS_PER_W, DV), jnp.float32),
            pltpu.SemaphoreType.DMA,
        ],
    )(gidx, vtab)

    return sums[:, :NUM_LABELS]


def kernel(features, emb_table, W, b):
    return _run(features, emb_table, W, b.astype(jnp.float32))


# speedup vs baseline: 2.3156x; 1.0705x over previous
import jax
import jax.numpy as jnp
from jax import lax
from jax.experimental import pallas as pl
from jax.experimental.pallas import tpu as pltpu
from jax.experimental.pallas import tpu_sc as plsc

VOCAB = 1000000
D = 64
HIST = 50
BATCH = 4096
NUM_LABELS = 2
DV = 16
NREG = 8

TBLK = 4096
NBLK = 31
REG = NBLK * TBLK
VPAD = NREG * REG

NC = 2
NS = 16
NW = NC * NS

BAGS_PER_W = BATCH // NW
IDX_PER_W = BAGS_PER_W * HIST


def _tc_build_table(x0, x1, x2, x3, x4, x5, x6, x7, w_ref, b_ref, o_ref):
    w = w_ref[...]
    parts = [
        lax.dot_general(x[...], w, (((0,), (1,)), ((), ())),
                        preferred_element_type=jnp.float32)
        for x in (x0, x1, x2, x3, x4, x5, x6, x7)
    ]
    o_ref[...] = jnp.concatenate(parts, axis=1) + b_ref[...]


def _sc_bag_sums(feat_hbm, table_hbm, out_hbm, idx_v, rows_v, sums_v, sem):
    wid = lax.axis_index("s") * NC + lax.axis_index("c")

    pltpu.sync_copy(feat_hbm.at[pl.ds(wid * IDX_PER_W, IDX_PER_W)], idx_v)
    pltpu.async_copy(table_hbm.at[idx_v], rows_v, sem).wait()

    zero = jnp.zeros((DV,), jnp.float32)

    def bag_body(bg, _):
        def r_body(r, acc):
            return acc + rows_v[bg * HIST + r, pl.ds(0, DV)]

        sums_v[bg, pl.ds(0, DV)] = lax.fori_loop(0, HIST, r_body, zero)
        return 0

    lax.fori_loop(0, BAGS_PER_W, bag_body, 0)

    pltpu.sync_copy(sums_v, out_hbm.at[pl.ds(wid * BAGS_PER_W, BAGS_PER_W)])


@jax.jit
def _run(features, emb_table, W, b):
    f = features.astype(jnp.int32).reshape(BATCH * HIST)
    gidx = (f % REG) * NREG + f // REG

    w16 = jnp.zeros((DV, D), jnp.float32).at[:NUM_LABELS].set(W) * (1.0 / HIST)
    b128 = jnp.tile(
        jnp.zeros((1, DV), jnp.float32).at[0, :NUM_LABELS].set(b) * (1.0 / HIST),
        (1, NREG))

    maxb = (VOCAB - 1) // TBLK
    region_spec = [
        pl.BlockSpec((D, TBLK),
                     lambda i, k=k: (0, jnp.minimum(k * NBLK + i, maxb)))
        for k in range(NREG)
    ]
    vtab = pl.pallas_call(
        _tc_build_table,
        grid=(NBLK,),
        in_specs=region_spec + [
            pl.BlockSpec((DV, D), lambda i: (0, 0)),
            pl.BlockSpec((1, NREG * DV), lambda i: (0, 0)),
        ],
        out_specs=pl.BlockSpec((TBLK, NREG * DV), lambda i: (i, 0)),
        out_shape=jax.ShapeDtypeStruct((REG, NREG * DV), jnp.float32),
    )(*([emb_table.T] * NREG), w16, b128)
    vtab = vtab.reshape(VPAD, DV)

    mesh = plsc.VectorSubcoreMesh(core_axis_name="c", subcore_axis_name="s",
                                  num_cores=NC, num_subcores=NS)
    sums = pl.kernel(
        _sc_bag_sums,
        out_type=jax.ShapeDtypeStruct((BATCH, DV), jnp.float32),
        mesh=mesh,
        compiler_params=pltpu.CompilerParams(use_tc_tiling_on_sc=False),
        scratch_types=[
            pltpu.VMEM((IDX_PER_W,), jnp.int32),
            pltpu.VMEM((IDX_PER_W, DV), jnp.float32),
            pltpu.VMEM((BAGS_PER_W, DV), jnp.float32),
            pltpu.SemaphoreType.DMA,
        ],
    )(gidx, vtab)

    return sums[:, :NUM_LABELS]


def kernel(features, emb_table, W, b):
    return _run(features, emb_table, W, b.astype(jnp.float32))
